# Initial kernel scaffold; baseline (speedup 1.0000x reference)
import jax
import jax.numpy as jnp
from jax.experimental import pallas as pl


def kernel(key, embedding_weight):
    raise NotImplementedError("write your pallas kernel here")


# speedup vs baseline: 2.9755x; 2.9755x over previous
import functools

import jax
import jax.numpy as jnp
from jax import lax
from jax.experimental import pallas as pl
from jax.experimental.pallas import tpu as pltpu
from jax.experimental.pallas import tpu_sc as plsc

D_MODEL = 128
B_TOTAL = 4096 * 50
NUM_WORKERS = 32
CHUNK = 128
CHUNKS_PER_W = B_TOTAL // (NUM_WORKERS * CHUNK)

_mesh = plsc.VectorSubcoreMesh(core_axis_name="c", subcore_axis_name="s")


@functools.partial(
    pl.kernel,
    mesh=_mesh,
    out_type=jax.ShapeDtypeStruct((B_TOTAL, D_MODEL), jnp.float32),
    scratch_types=[
        pltpu.VMEM((CHUNKS_PER_W, CHUNK), jnp.int32),
        pltpu.VMEM((CHUNK, D_MODEL), jnp.float32),
        pltpu.SemaphoreType.DMA,
    ],
)
def _gather_kernel(table_hbm, idx_hbm, out_hbm, idx_v, rows_v, sem):
    wid = lax.axis_index("s") * 2 + lax.axis_index("c")
    row0 = wid * CHUNKS_PER_W
    pltpu.sync_copy(idx_hbm.at[wid], idx_v)

    def body(g, carry):
        pltpu.async_copy(table_hbm.at[idx_v.at[g]], rows_v, sem).wait()
        pltpu.sync_copy(rows_v, out_hbm.at[pl.ds((row0 + g) * CHUNK, CHUNK)])
        return carry

    lax.fori_loop(0, CHUNKS_PER_W, body, None)


def kernel(key, embedding_weight):
    idx = key.reshape(NUM_WORKERS, CHUNKS_PER_W, CHUNK).astype(jnp.int32)
    out = _gather_kernel(embedding_weight, idx)
    return out.reshape(key.shape + (D_MODEL,))


# speedup vs baseline: 3.3542x; 1.1273x over previous
import functools

import jax
import jax.numpy as jnp
from jax import lax
from jax.experimental import pallas as pl
from jax.experimental.pallas import tpu as pltpu
from jax.experimental.pallas import tpu_sc as plsc

D_MODEL = 128
B_TOTAL = 4096 * 50
NUM_WORKERS = 32
CHUNK = 128
CHUNKS_PER_W = B_TOTAL // (NUM_WORKERS * CHUNK)

_mesh = plsc.VectorSubcoreMesh(core_axis_name="c", subcore_axis_name="s")

NBUF = 6
LAG = 3


@functools.partial(
    pl.kernel,
    mesh=_mesh,
    out_type=jax.ShapeDtypeStruct((B_TOTAL, D_MODEL), jnp.float32),
    scratch_types=[
        pltpu.VMEM((CHUNKS_PER_W, CHUNK), jnp.int32),
        pltpu.VMEM((NBUF, CHUNK, D_MODEL), jnp.float32),
        pltpu.SemaphoreType.DMA((NBUF,)),
        pltpu.SemaphoreType.DMA((NBUF,)),
    ],
)
def _gather_kernel(table_hbm, idx_hbm, out_hbm, idx_v, rows_v, gsem, ssem):
    wid = lax.axis_index("s") * 2 + lax.axis_index("c")
    row0 = wid * CHUNKS_PER_W
    pltpu.sync_copy(idx_hbm.at[wid], idx_v)

    n = CHUNKS_PER_W

    def body(g, carry):
        b = lax.rem(g, NBUF)

        @pl.when(g >= NBUF)
        def _():
            pltpu.make_async_copy(
                rows_v.at[b], out_hbm.at[pl.ds((row0 + g - NBUF) * CHUNK, CHUNK)],
                ssem.at[b]).wait()

        @pl.when(g < n)
        def _():
            pltpu.async_copy(table_hbm.at[idx_v.at[g]], rows_v.at[b], gsem.at[b])

        h = g - LAG
        bh = lax.rem(g + (NBUF - LAG), NBUF)

        @pl.when((g >= LAG) & (h < n))
        def _():
            pltpu.make_async_copy(
                table_hbm.at[idx_v.at[h]], rows_v.at[bh], gsem.at[bh]).wait()
            pltpu.async_copy(
                rows_v.at[bh], out_hbm.at[pl.ds((row0 + h) * CHUNK, CHUNK)],
                ssem.at[bh])

        return carry

    lax.fori_loop(0, n + NBUF, body, None)


def kernel(key, embedding_weight):
    idx = key.reshape(NUM_WORKERS, CHUNKS_PER_W, CHUNK).astype(jnp.int32)
    out = _gather_kernel(embedding_weight, idx)
    return out.reshape(key.shape + (D_MODEL,))


# speedup vs baseline: 5.9913x; 1.7862x over previous
import functools

import jax
import jax.numpy as jnp
from jax import lax
from jax.experimental import pallas as pl
from jax.experimental.pallas import tpu as pltpu
from jax.experimental.pallas import tpu_sc as plsc

BATCH = 4096
HIST = 50
D_MODEL = 128
NUM_WORKERS = 32
ROWS_PER_W = BATCH // NUM_WORKERS
GROUP = 4
STEPS = ROWS_PER_W // GROUP
NBUF = 4
LAG = 2

_mesh = plsc.VectorSubcoreMesh(core_axis_name="c", subcore_axis_name="s")


@functools.partial(
    pl.kernel,
    mesh=_mesh,
    out_type=jax.ShapeDtypeStruct((BATCH, HIST, D_MODEL), jnp.float32),
    scratch_types=[
        pltpu.VMEM((ROWS_PER_W, HIST), jnp.int32),
        pltpu.VMEM((NBUF, GROUP, HIST, D_MODEL), jnp.float32),
        pltpu.SemaphoreType.DMA((NBUF,)),
        pltpu.SemaphoreType.DMA((NBUF,)),
    ],
)
def _gather_kernel(table_hbm, idx_hbm, out_hbm, idx_v, rows_v, gsem, ssem):
    wid = lax.axis_index("s") * 2 + lax.axis_index("c")
    batch0 = wid * ROWS_PER_W
    pltpu.sync_copy(idx_hbm.at[pl.ds(batch0, ROWS_PER_W)], idx_v)

    def body(i, carry):
        b = lax.rem(i, NBUF)

        @pl.when(i >= NBUF)
        def _():
            pltpu.make_async_copy(
                rows_v.at[b],
                out_hbm.at[pl.ds(batch0 + (i - NBUF) * GROUP, GROUP)],
                ssem.at[b]).wait()

        @pl.when(i < STEPS)
        def _():
            for k in range(GROUP):
                pltpu.async_copy(
                    table_hbm.at[idx_v.at[i * GROUP + k]],
                    rows_v.at[b, k], gsem.at[b])

        h = i - LAG
        bh = lax.rem(i + (NBUF - LAG), NBUF)

        @pl.when((i >= LAG) & (h < STEPS))
        def _():
            for k in range(GROUP):
                pltpu.make_async_copy(
                    table_hbm.at[idx_v.at[h * GROUP + k]],
                    rows_v.at[bh, k], gsem.at[bh]).wait()
            pltpu.async_copy(
                rows_v.at[bh],
                out_hbm.at[pl.ds(batch0 + h * GROUP, GROUP)],
                ssem.at[bh])

        return carry

    lax.fori_loop(0, STEPS + NBUF, body, None)


def kernel(key, embedding_weight):
    return _gather_kernel(embedding_weight, key.astype(jnp.int32))


# speedup vs baseline: 10.7898x; 1.8009x over previous
import functools

import jax
import jax.numpy as jnp
from jax import lax
from jax.experimental import pallas as pl
from jax.experimental.pallas import tpu as pltpu
from jax.experimental.pallas import tpu_sc as plsc

BATCH = 4096
HIST = 50
D_MODEL = 128
NUM_WORKERS = 32
BPW = BATCH // NUM_WORKERS
NBUF = 6
LAG = 3

_mesh = plsc.VectorSubcoreMesh(core_axis_name="c", subcore_axis_name="s")


@functools.partial(
    pl.kernel,
    mesh=_mesh,
    out_type=jax.ShapeDtypeStruct((HIST, BATCH, D_MODEL), jnp.float32),
    scratch_types=[
        pltpu.VMEM((HIST, BPW), jnp.int32),
        pltpu.VMEM((NBUF, BPW, D_MODEL), jnp.float32),
        pltpu.SemaphoreType.DMA((NBUF,)),
        pltpu.SemaphoreType.DMA((NBUF,)),
    ],
)
def _gather_kernel(table_hbm, idx_hbm, out_hbm, idx_v, rows_v, gsem, ssem):
    wid = lax.axis_index("s") * 2 + lax.axis_index("c")
    batch0 = wid * BPW
    pltpu.sync_copy(idx_hbm.at[:, pl.ds(batch0, BPW)], idx_v)

    def body(i, carry):
        b = lax.rem(i, NBUF)

        @pl.when(i >= NBUF)
        def _():
            pltpu.make_async_copy(
                rows_v.at[b],
                out_hbm.at[i - NBUF, pl.ds(batch0, BPW)],
                ssem.at[b]).wait()

        @pl.when(i < HIST)
        def _():
            pltpu.async_copy(
                table_hbm.at[idx_v.at[i]], rows_v.at[b], gsem.at[b])

        h = i - LAG
        bh = lax.rem(i + (NBUF - LAG), NBUF)

        @pl.when((i >= LAG) & (h < HIST))
        def _():
            pltpu.make_async_copy(
                table_hbm.at[idx_v.at[h]], rows_v.at[bh], gsem.at[bh]).wait()
            pltpu.async_copy(
                rows_v.at[bh],
                out_hbm.at[h, pl.ds(batch0, BPW)],
                ssem.at[bh])

        return carry

    lax.fori_loop(0, HIST + NBUF, body, None)


def kernel(key, embedding_weight):
    idx_t = key.astype(jnp.int32).T
    out_t = _gather_kernel(embedding_weight, idx_t)
    return jnp.transpose(out_t, (1, 0, 2))


# speedup vs baseline: 10.8565x; 1.0062x over previous
import functools

import jax
import jax.numpy as jnp
from jax import lax
from jax.experimental import pallas as pl
from jax.experimental.pallas import tpu as pltpu
from jax.experimental.pallas import tpu_sc as plsc

BATCH = 4096
HIST = 50
D_MODEL = 128
NUM_WORKERS = 32
BPW = BATCH // NUM_WORKERS
NBUF = 7
LAG = 4

_mesh = plsc.VectorSubcoreMesh(core_axis_name="c", subcore_axis_name="s")


@functools.partial(
    pl.kernel,
    mesh=_mesh,
    out_type=jax.ShapeDtypeStruct((HIST, BATCH, D_MODEL), jnp.float32),
    scratch_types=[
        pltpu.VMEM((HIST, BPW), jnp.int32),
        pltpu.VMEM((NBUF, BPW, D_MODEL), jnp.float32),
        pltpu.SemaphoreType.DMA((NBUF,)),
        pltpu.SemaphoreType.DMA((NBUF,)),
    ],
)
def _gather_kernel(table_hbm, idx_hbm, out_hbm, idx_v, rows_v, gsem, ssem):
    wid = lax.axis_index("s") * 2 + lax.axis_index("c")
    batch0 = wid * BPW
    pltpu.sync_copy(idx_hbm.at[:, pl.ds(batch0, BPW)], idx_v)

    def body(i, carry):
        b = lax.rem(i, NBUF)

        @pl.when(i >= NBUF)
        def _():
            pltpu.make_async_copy(
                rows_v.at[b],
                out_hbm.at[i - NBUF, pl.ds(batch0, BPW)],
                ssem.at[b]).wait()

        @pl.when(i < HIST)
        def _():
            pltpu.async_copy(
                table_hbm.at[idx_v.at[i]], rows_v.at[b], gsem.at[b])

        h = i - LAG
        bh = lax.rem(i + (NBUF - LAG), NBUF)

        @pl.when((i >= LAG) & (h < HIST))
        def _():
            pltpu.make_async_copy(
                table_hbm.at[idx_v.at[h]], rows_v.at[bh], gsem.at[bh]).wait()
            pltpu.async_copy(
                rows_v.at[bh],
                out_hbm.at[h, pl.ds(batch0, BPW)],
                ssem.at[bh])

        return carry

    lax.fori_loop(0, HIST + NBUF, body, None)


def kernel(key, embedding_weight):
    idx_t = key.astype(jnp.int32).T
    out_t = _gather_kernel(embedding_weight, idx_t)
    return jnp.transpose(out_t, (1, 0, 2))


# speedup vs baseline: 10.8624x; 1.0005x over previous
import functools

import jax
import jax.numpy as jnp
from jax import lax
from jax.experimental import pallas as pl
from jax.experimental.pallas import tpu as pltpu
from jax.experimental.pallas import tpu_sc as plsc

BATCH = 4096
HIST = 50
D_MODEL = 128
NUM_WORKERS = 32
BPW = BATCH // NUM_WORKERS
NBUF = 7
LAG = 5

_mesh = plsc.VectorSubcoreMesh(core_axis_name="c", subcore_axis_name="s")


@functools.partial(
    pl.kernel,
    mesh=_mesh,
    out_type=jax.ShapeDtypeStruct((HIST, BATCH, D_MODEL), jnp.float32),
    scratch_types=[
        pltpu.VMEM((HIST, BPW), jnp.int32),
        pltpu.VMEM((NBUF, BPW, D_MODEL), jnp.float32),
        pltpu.SemaphoreType.DMA((NBUF,)),
        pltpu.SemaphoreType.DMA((NBUF,)),
    ],
)
def _gather_kernel(table_hbm, idx_hbm, out_hbm, idx_v, rows_v, gsem, ssem):
    wid = lax.axis_index("s") * 2 + lax.axis_index("c")
    batch0 = wid * BPW
    pltpu.sync_copy(idx_hbm.at[:, pl.ds(batch0, BPW)], idx_v)

    def body(i, carry):
        b = lax.rem(i, NBUF)

        @pl.when(i >= NBUF)
        def _():
            pltpu.make_async_copy(
                rows_v.at[b],
                out_hbm.at[i - NBUF, pl.ds(batch0, BPW)],
                ssem.at[b]).wait()

        @pl.when(i < HIST)
        def _():
            pltpu.async_copy(
                table_hbm.at[idx_v.at[i]], rows_v.at[b], gsem.at[b])

        h = i - LAG
        bh = lax.rem(i + (NBUF - LAG), NBUF)

        @pl.when((i >= LAG) & (h < HIST))
        def _():
            pltpu.make_async_copy(
                table_hbm.at[idx_v.at[h]], rows_v.at[bh], gsem.at[bh]).wait()
            pltpu.async_copy(
                rows_v.at[bh],
                out_hbm.at[h, pl.ds(batch0, BPW)],
                ssem.at[bh])

        return carry

    lax.fori_loop(0, HIST + NBUF, body, None)


def kernel(key, embedding_weight):
    idx_t = key.astype(jnp.int32).T
    out_t = _gather_kernel(embedding_weight, idx_t)
    return jnp.transpose(out_t, (1, 0, 2))
